# baseline (speedup 1.0000x reference)
import functools

import jax
import jax.numpy as jnp
import numpy as np
from jax.experimental import pallas as pl
from jax.experimental.pallas import tpu as pltpu

_IMAGE_SIZE = 1024
_STRIDE = 16
_SCALES = (128.0, 256.0, 512.0)
_RATIOS = (0.5, 1.0, 2.0)
_LANES = 128
_BLK_R = 96


def _anchor_planes(A):
    fs = _IMAGE_SIZE // _STRIDE
    shifts = ((np.arange(fs, dtype=np.float32) + np.float32(0.5))
              * np.float32(_STRIDE))
    cy, cx = np.meshgrid(shifts, shifts, indexing="ij")
    centers = np.stack([cx.ravel(), cy.ravel()], axis=1)
    ws, hs = [], []
    for s in _SCALES:
        for r in _RATIOS:
            ws.append(s * np.sqrt(r))
            hs.append(s / np.sqrt(r))
    wh = np.stack([ws, hs], axis=1).astype(np.float32)
    ctr = np.repeat(centers, wh.shape[0], axis=0)
    whr = np.tile(wh, (centers.shape[0], 1))
    x1y1 = ctr - whr / np.float32(2.0)
    x2y2 = ctr + whr / np.float32(2.0)
    ax1, ay1 = x1y1[:, 0], x1y1[:, 1]
    ax2, ay2 = x2y2[:, 0], x2y2[:, 1]
    aw = ax2 - ax1
    ah = ay2 - ay1
    acx = (ax1 + ax2) * np.float32(0.5)
    acy = (ay1 + ay2) * np.float32(0.5)
    area = aw * ah
    planes = [ax1, ay1, ax2, ay2, area, acx, acy,
              1.0 / aw, 1.0 / ah, np.log(aw), np.log(ah)]
    R = A // _LANES
    return [jnp.asarray(p.astype(np.float32).reshape(R, _LANES)) for p in planes]


def _body(gt_ref, po_hbm, plab_hbm,
          ax1_ref, ay1_ref, ax2_ref, ay2_ref, area_ref, acx_ref, acy_ref,
          iw_ref, ih_ref, law_ref, lah_ref,
          loss_ref, box_ref, cls_ref,
          acc_ref, po_buf, pl_buf, sem_po, sem_pl,
          *, nb, nj, ng, denom_cls):
    b = pl.program_id(0)
    j = pl.program_id(1)
    k = b * nj + j
    slot = jax.lax.rem(k, 2)
    rows = pl.ds(j * _BLK_R, _BLK_R)

    def start(i, s):
        bi = i // nj
        ji = i - bi * nj
        pltpu.make_async_copy(
            po_hbm.at[bi, pl.ds(ji * 4 * _BLK_R, 4 * _BLK_R), :],
            po_buf.at[s], sem_po.at[s]).start()
        pltpu.make_async_copy(
            plab_hbm.at[bi, pl.ds(ji * _BLK_R, _BLK_R), :],
            pl_buf.at[s], sem_pl.at[s]).start()

    @pl.when(k == 0)
    def _():
        start(k, slot)

    @pl.when(k + 1 < nb * nj)
    def _():
        start(k + 1, 1 - slot)

    ax1 = ax1_ref[rows, :]
    ay1 = ay1_ref[rows, :]
    ax2 = ax2_ref[rows, :]
    ay2 = ay2_ref[rows, :]
    area_a = area_ref[rows, :]

    shp = ax1.shape
    best = jnp.full(shp, -1.0, dtype=jnp.float32)
    mcx = jnp.zeros(shp, dtype=jnp.float32)
    mcy = jnp.zeros(shp, dtype=jnp.float32)
    mw = jnp.zeros(shp, dtype=jnp.float32)
    mh = jnp.zeros(shp, dtype=jnp.float32)

    def gstep(g, carry):
        best, mcx, mcy, mw, mh = carry
        bx1 = gt_ref[0, g, 0]
        by1 = gt_ref[0, g, 1]
        bx2 = gt_ref[0, g, 2]
        by2 = gt_ref[0, g, 3]
        ix1 = jnp.maximum(ax1, bx1)
        iy1 = jnp.maximum(ay1, by1)
        ix2 = jnp.minimum(ax2, bx2)
        iy2 = jnp.minimum(ay2, by2)
        inter = jnp.maximum(ix2 - ix1, 0.0) * jnp.maximum(iy2 - iy1, 0.0)
        area_b = (bx2 - bx1) * (by2 - by1)
        union = jnp.maximum(area_a + area_b - inter, 1e-8)
        iou = inter / union
        upd = iou > best
        best = jnp.maximum(iou, best)
        mcx = jnp.where(upd, (bx1 + bx2) * 0.5, mcx)
        mcy = jnp.where(upd, (by1 + by2) * 0.5, mcy)
        mw = jnp.where(upd, bx2 - bx1, mw)
        mh = jnp.where(upd, by2 - by1, mh)
        return best, mcx, mcy, mw, mh

    best, mcx, mcy, mw, mh = jax.lax.fori_loop(
        0, ng, gstep, (best, mcx, mcy, mw, mh), unroll=True)

    pos = best >= 0.5
    ocx = (mcx - acx_ref[rows, :]) * iw_ref[rows, :]
    ocy = (mcy - acy_ref[rows, :]) * ih_ref[rows, :]
    ow = jnp.log(jnp.maximum(mw, 1e-6)) - law_ref[rows, :]
    oh = jnp.log(jnp.maximum(mh, 1e-6)) - lah_ref[rows, :]

    pltpu.make_async_copy(
        po_hbm.at[b, pl.ds(j * 4 * _BLK_R, 4 * _BLK_R), :],
        po_buf.at[slot], sem_po.at[slot]).wait()
    pltpu.make_async_copy(
        plab_hbm.at[b, pl.ds(j * _BLK_R, _BLK_R), :],
        pl_buf.at[slot], sem_pl.at[slot]).wait()

    po4 = po_buf[slot].reshape(_BLK_R, 4, _LANES)
    d = (jnp.abs(po4[:, 0, :] - ocx) + jnp.abs(po4[:, 1, :] - ocy) +
         jnp.abs(po4[:, 2, :] - ow) + jnp.abs(po4[:, 3, :] - oh))
    posf = pos.astype(jnp.float32)
    s_box = jnp.sum(jnp.where(pos, d, 0.0))
    s_pos = jnp.sum(posf)

    x = pl_buf[slot]
    bce = jnp.maximum(x, 0.0) - x * posf + jnp.log1p(jnp.exp(-jnp.abs(x)))
    s_bce = jnp.sum(bce)

    @pl.when(k == 0)
    def _():
        acc_ref[0] = 0.0
        acc_ref[1] = 0.0
        acc_ref[2] = 0.0

    acc_ref[0] += s_pos
    acc_ref[1] += s_box
    acc_ref[2] += s_bce

    @pl.when(k == nb * nj - 1)
    def _():
        box_loss = acc_ref[1] / jnp.maximum(acc_ref[0] * 4.0, 1.0)
        cls_loss = acc_ref[2] * denom_cls
        loss_ref[0, 0] = box_loss + cls_loss
        box_ref[0, 0] = box_loss
        cls_ref[0, 0] = cls_loss


@jax.jit
def kernel(predicted_labels, predicted_offsets, gt_boxes):
    B, A, _ = predicted_labels.shape
    G = gt_boxes.shape[1]
    R = A // _LANES
    NJ = R // _BLK_R
    planes = _anchor_planes(A)

    po = (predicted_offsets.reshape(B, R, _LANES, 4)
          .transpose(0, 1, 3, 2).reshape(B, R * 4, _LANES))
    plab = predicted_labels.reshape(B, R, _LANES)

    plane_spec = pl.BlockSpec((R, _LANES), lambda b, j: (0, 0))
    out_spec = pl.BlockSpec(memory_space=pltpu.SMEM)
    body = functools.partial(_body, nb=B, nj=NJ, ng=G,
                             denom_cls=1.0 / float(B * A))
    outs = pl.pallas_call(
        body,
        grid=(B, NJ),
        in_specs=[
            pl.BlockSpec((1, G, 4), lambda b, j: (b, 0, 0),
                         memory_space=pltpu.SMEM),
            pl.BlockSpec(memory_space=pl.ANY),
            pl.BlockSpec(memory_space=pl.ANY),
        ] + [plane_spec] * 11,
        out_specs=[out_spec, out_spec, out_spec],
        out_shape=[jax.ShapeDtypeStruct((1, 1), jnp.float32)] * 3,
        scratch_shapes=[
            pltpu.SMEM((3,), jnp.float32),
            pltpu.VMEM((2, 4 * _BLK_R, _LANES), jnp.float32),
            pltpu.VMEM((2, _BLK_R, _LANES), jnp.float32),
            pltpu.SemaphoreType.DMA((2,)),
            pltpu.SemaphoreType.DMA((2,)),
        ],
    )(gt_boxes, po, plab, *planes)
    loss, box_loss, cls_loss = (o[0, 0] for o in outs)
    return (loss, box_loss, cls_loss)


# speedup vs baseline: 1.1024x; 1.1024x over previous
import functools

import jax
import jax.numpy as jnp
import numpy as np
from jax.experimental import pallas as pl
from jax.experimental.pallas import tpu as pltpu

_IMAGE_SIZE = 1024
_STRIDE = 16
_SCALES = (128.0, 256.0, 512.0)
_RATIOS = (0.5, 1.0, 2.0)
_LANES = 128
_BLK_R = 96


def _anchor_planes(A):
    fs = _IMAGE_SIZE // _STRIDE
    shifts = ((np.arange(fs, dtype=np.float32) + np.float32(0.5))
              * np.float32(_STRIDE))
    cy, cx = np.meshgrid(shifts, shifts, indexing="ij")
    centers = np.stack([cx.ravel(), cy.ravel()], axis=1)
    ws, hs = [], []
    for s in _SCALES:
        for r in _RATIOS:
            ws.append(s * np.sqrt(r))
            hs.append(s / np.sqrt(r))
    wh = np.stack([ws, hs], axis=1).astype(np.float32)
    ctr = np.repeat(centers, wh.shape[0], axis=0)
    whr = np.tile(wh, (centers.shape[0], 1))
    x1y1 = ctr - whr / np.float32(2.0)
    x2y2 = ctr + whr / np.float32(2.0)
    ax1, ay1 = x1y1[:, 0], x1y1[:, 1]
    ax2, ay2 = x2y2[:, 0], x2y2[:, 1]
    aw = ax2 - ax1
    ah = ay2 - ay1
    acx = (ax1 + ax2) * np.float32(0.5)
    acy = (ay1 + ay2) * np.float32(0.5)
    area = aw * ah
    planes = [ax1, ay1, ax2, ay2, area, acx, acy,
              1.0 / aw, 1.0 / ah, np.log(aw), np.log(ah)]
    R = A // _LANES
    return [jnp.asarray(p.astype(np.float32).reshape(R, _LANES)) for p in planes]


def _body(gt_ref, po_ref, plab_ref,
          ax1_ref, ay1_ref, ax2_ref, ay2_ref, area_ref, acx_ref, acy_ref,
          iw_ref, ih_ref, law_ref, lah_ref,
          loss_ref, box_ref, cls_ref, acc_ref, *, nb, nj, ng, denom_cls):
    b = pl.program_id(0)
    j = pl.program_id(1)
    rows = pl.ds(j * _BLK_R, _BLK_R)

    ax1 = ax1_ref[rows, :]
    ay1 = ay1_ref[rows, :]
    ax2 = ax2_ref[rows, :]
    ay2 = ay2_ref[rows, :]
    area_a = area_ref[rows, :]

    shp = ax1.shape
    best = jnp.full(shp, -1.0, dtype=jnp.float32)
    mcx = jnp.zeros(shp, dtype=jnp.float32)
    mcy = jnp.zeros(shp, dtype=jnp.float32)
    mw = jnp.zeros(shp, dtype=jnp.float32)
    mh = jnp.zeros(shp, dtype=jnp.float32)

    def gstep(g, carry):
        best, mcx, mcy, mw, mh = carry
        bx1 = gt_ref[0, g, 0]
        by1 = gt_ref[0, g, 1]
        bx2 = gt_ref[0, g, 2]
        by2 = gt_ref[0, g, 3]
        ix1 = jnp.maximum(ax1, bx1)
        iy1 = jnp.maximum(ay1, by1)
        ix2 = jnp.minimum(ax2, bx2)
        iy2 = jnp.minimum(ay2, by2)
        inter = jnp.maximum(ix2 - ix1, 0.0) * jnp.maximum(iy2 - iy1, 0.0)
        area_b = (bx2 - bx1) * (by2 - by1)
        union = jnp.maximum(area_a + area_b - inter, 1e-8)
        iou = inter / union
        upd = iou > best
        best = jnp.maximum(iou, best)
        mcx = jnp.where(upd, (bx1 + bx2) * 0.5, mcx)
        mcy = jnp.where(upd, (by1 + by2) * 0.5, mcy)
        mw = jnp.where(upd, bx2 - bx1, mw)
        mh = jnp.where(upd, by2 - by1, mh)
        return best, mcx, mcy, mw, mh

    best, mcx, mcy, mw, mh = jax.lax.fori_loop(
        0, ng, gstep, (best, mcx, mcy, mw, mh), unroll=True)

    pos = best >= 0.5
    ocx = (mcx - acx_ref[rows, :]) * iw_ref[rows, :]
    ocy = (mcy - acy_ref[rows, :]) * ih_ref[rows, :]
    ow = jnp.log(jnp.maximum(mw, 1e-6)) - law_ref[rows, :]
    oh = jnp.log(jnp.maximum(mh, 1e-6)) - lah_ref[rows, :]
    d = (jnp.abs(po_ref[0, 0] - ocx) + jnp.abs(po_ref[0, 1] - ocy) +
         jnp.abs(po_ref[0, 2] - ow) + jnp.abs(po_ref[0, 3] - oh))
    posf = pos.astype(jnp.float32)
    s_box = jnp.sum(jnp.where(pos, d, 0.0))
    s_pos = jnp.sum(posf)

    x = plab_ref[0]
    bce = jnp.maximum(x, 0.0) - x * posf + jnp.log1p(jnp.exp(-jnp.abs(x)))
    s_bce = jnp.sum(bce)

    @pl.when(jnp.logical_and(b == 0, j == 0))
    def _():
        acc_ref[0] = 0.0
        acc_ref[1] = 0.0
        acc_ref[2] = 0.0

    acc_ref[0] += s_pos
    acc_ref[1] += s_box
    acc_ref[2] += s_bce

    @pl.when(jnp.logical_and(b == nb - 1, j == nj - 1))
    def _():
        box_loss = acc_ref[1] / jnp.maximum(acc_ref[0] * 4.0, 1.0)
        cls_loss = acc_ref[2] * denom_cls
        loss_ref[0, 0] = box_loss + cls_loss
        box_ref[0, 0] = box_loss
        cls_ref[0, 0] = cls_loss


@jax.jit
def kernel(predicted_labels, predicted_offsets, gt_boxes):
    B, A, _ = predicted_labels.shape
    G = gt_boxes.shape[1]
    R = A // _LANES
    NJ = R // _BLK_R
    planes = _anchor_planes(A)

    po = predicted_offsets.transpose(0, 2, 1).reshape(B, 4, R, _LANES)
    plab = predicted_labels.reshape(B, R, _LANES)

    plane_spec = pl.BlockSpec((R, _LANES), lambda b, j: (0, 0))
    out_spec = pl.BlockSpec(memory_space=pltpu.SMEM)
    body = functools.partial(_body, nb=B, nj=NJ, ng=G,
                             denom_cls=1.0 / float(B * A))
    outs = pl.pallas_call(
        body,
        grid=(B, NJ),
        in_specs=[
            pl.BlockSpec((1, G, 4), lambda b, j: (b, 0, 0),
                         memory_space=pltpu.SMEM),
            pl.BlockSpec((1, 4, _BLK_R, _LANES), lambda b, j: (b, 0, j, 0)),
            pl.BlockSpec((1, _BLK_R, _LANES), lambda b, j: (b, j, 0)),
        ] + [plane_spec] * 11,
        out_specs=[out_spec, out_spec, out_spec],
        out_shape=[jax.ShapeDtypeStruct((1, 1), jnp.float32)] * 3,
        scratch_shapes=[pltpu.SMEM((3,), jnp.float32)],
    )(gt_boxes, po, plab, *planes)
    loss, box_loss, cls_loss = (o[0, 0] for o in outs)
    return (loss, box_loss, cls_loss)
